# Initial kernel scaffold; baseline (speedup 1.0000x reference)
import jax
import jax.numpy as jnp
from jax.experimental import pallas as pl


def kernel(point_coord, features):
    raise NotImplementedError("write your pallas kernel here")


# speedup vs baseline: 9.6386x; 9.6386x over previous
import functools

import jax
import jax.numpy as jnp
from jax import lax
from jax.experimental import pallas as pl
from jax.experimental.pallas import tpu as pltpu
from jax.experimental.pallas import tpu_sc as plsc

B = 8
N = 16384
S = 2048
L = 16
NC = 2
NS = 16
TPB = 4
NPT = N // TPB
CH = NPT // L
BIG = 1 << 30


def _fps_body(xs_hbm, ys_hbm, zs_hbm, out_hbm,
              xv, yv, zv, dv, stage, comb, outrows, shared):
    c = lax.axis_index("c")
    s = lax.axis_index("s")
    batch = c * (NS // TPB) + s // TPB
    part = s % TPB
    base = part * NPT
    group = (s // TPB) * TPB

    pltpu.sync_copy(xs_hbm.at[batch, pl.ds(base, NPT)], xv)
    pltpu.sync_copy(ys_hbm.at[batch, pl.ds(base, NPT)], yv)
    pltpu.sync_copy(zs_hbm.at[batch, pl.ds(base, NPT)], zv)

    inf16 = jnp.full((L,), jnp.inf, dtype=jnp.float32)

    def fill(k, carry):
        dv[pl.ds(k * L, L)] = inf16
        return carry

    lax.fori_loop(0, CH, fill, 0)

    pltpu.sync_copy(xs_hbm.at[batch, pl.ds(0, L)], stage)
    px0 = stage[...][0]
    pltpu.sync_copy(ys_hbm.at[batch, pl.ds(0, L)], stage)
    py0 = stage[...][0]
    pltpu.sync_copy(zs_hbm.at[batch, pl.ds(0, L)], stage)
    pz0 = stage[...][0]

    lane = lax.iota(jnp.int32, L)

    @pl.when(part == 0)
    def _():
        row0 = jnp.where(lane == 2, jnp.full((L,), px0, dtype=jnp.float32),
                         jnp.where(lane == 3,
                                   jnp.full((L,), py0, dtype=jnp.float32),
                                   jnp.full((L,), pz0, dtype=jnp.float32)))
        outrows[0] = row0

    gi_base = lane + base
    neginf16 = jnp.full((L,), -jnp.inf, dtype=jnp.float32)
    zero_i16 = jnp.zeros((L,), dtype=jnp.int32)

    def step(i, carry):
        px, py, pz = carry
        pxv = jnp.full((L,), px, dtype=jnp.float32)
        pyv = jnp.full((L,), py, dtype=jnp.float32)
        pzv = jnp.full((L,), pz, dtype=jnp.float32)

        def chunk(k, st):
            bestv, besti = st
            off = k * L
            dx = xv[pl.ds(off, L)] - pxv
            dy = yv[pl.ds(off, L)] - pyv
            dz = zv[pl.ds(off, L)] - pzv
            d = dx * dx + dy * dy + dz * dz
            dn = jnp.minimum(dv[pl.ds(off, L)], d)
            dv[pl.ds(off, L)] = dn
            m = dn > bestv
            bestv = jnp.where(m, dn, bestv)
            besti = jnp.where(m, gi_base + off, besti)
            return bestv, besti

        bestv, besti = lax.fori_loop(0, CH, chunk, (neginf16, zero_i16),
                                     unroll=8)

        vm = jnp.max(bestv)
        cand = jnp.where(bestv == jnp.full((L,), vm, dtype=jnp.float32),
                         besti, jnp.full((L,), BIG, dtype=jnp.int32))
        gidx = jnp.min(cand)
        liv = jnp.full((L,), gidx - base, dtype=jnp.int32)
        wx = plsc.load_gather(xv, [liv])
        wy = plsc.load_gather(yv, [liv])
        wz = plsc.load_gather(zv, [liv])

        gf = jnp.full((L,), gidx.astype(jnp.float32), dtype=jnp.float32)
        vmv = jnp.full((L,), vm, dtype=jnp.float32)
        stv = jnp.where(lane == 0, vmv,
                        jnp.where(lane == 1, gf,
                                  jnp.where(lane == 2, wx,
                                            jnp.where(lane == 3, wy, wz))))
        stage[...] = stv

        pltpu.sync_copy(stage, shared.at[s])
        plsc.subcore_barrier()
        pltpu.sync_copy(shared.at[pl.ds(group, TPB)], comb)
        plsc.subcore_barrier()

        w = comb[0]
        bv = w[0]
        for j in range(1, TPB):
            cj = comb[j]
            v = cj[0]
            better = v > bv
            bv = jnp.where(better, v, bv)
            w = jnp.where(better, cj, w)

        @pl.when(part == 0)
        def _():
            outrows[i] = w

        return w[2], w[3], w[4]

    lax.fori_loop(1, S, step, (px0, py0, pz0))

    @pl.when(part == 0)
    def _():
        pltpu.sync_copy(outrows, out_hbm.at[batch])


_fps_sc = functools.partial(
    pl.kernel,
    out_type=jax.ShapeDtypeStruct((B, S, L), jnp.float32),
    mesh=plsc.VectorSubcoreMesh(core_axis_name="c", subcore_axis_name="s",
                                num_cores=NC, num_subcores=NS),
    scratch_types=[
        pltpu.VMEM((NPT,), jnp.float32),
        pltpu.VMEM((NPT,), jnp.float32),
        pltpu.VMEM((NPT,), jnp.float32),
        pltpu.VMEM((NPT,), jnp.float32),
        pltpu.VMEM((L,), jnp.float32),
        pltpu.VMEM((TPB, L), jnp.float32),
        pltpu.VMEM((S, L), jnp.float32),
        pltpu.VMEM_SHARED((NS, L), jnp.float32),
    ],
    compiler_params=pltpu.CompilerParams(needs_layout_passes=False,
                                         use_tc_tiling_on_sc=False),
)(_fps_body)


@jax.jit
def kernel(point_coord, features):
    del features
    xs = point_coord[:, :, 0]
    ys = point_coord[:, :, 1]
    zs = point_coord[:, :, 2]
    out = _fps_sc(xs, ys, zs)
    return out[:, :, 2:5]


# speedup vs baseline: 28.0656x; 2.9118x over previous
import functools

import jax
import jax.numpy as jnp
from jax import lax
from jax.experimental import pallas as pl
from jax.experimental.pallas import tpu as pltpu
from jax.experimental.pallas import tpu_sc as plsc

B = 8
N = 16384
S = 2048
L = 16
NC = 2
NS = 16
TPB = 4
NPT = N // TPB
CH = NPT // L
BIG = 1 << 30


def _fps_body(xs_hbm, ys_hbm, zs_hbm, out_hbm,
              xv, yv, zv, dv, stage, comb, outrows, shared):
    c = lax.axis_index("c")
    s = lax.axis_index("s")
    batch = c * (NS // TPB) + s // TPB
    part = s % TPB
    base = part * NPT
    group = (s // TPB) * TPB

    pltpu.sync_copy(xs_hbm.at[batch, pl.ds(base, NPT)], xv)
    pltpu.sync_copy(ys_hbm.at[batch, pl.ds(base, NPT)], yv)
    pltpu.sync_copy(zs_hbm.at[batch, pl.ds(base, NPT)], zv)

    inf16 = jnp.full((L,), jnp.inf, dtype=jnp.float32)

    def fill(k, carry):
        dv[pl.ds(k * L, L)] = inf16
        return carry

    lax.fori_loop(0, CH, fill, 0)

    pltpu.sync_copy(xs_hbm.at[batch, pl.ds(0, L)], stage)
    px0 = stage[...][0]
    pltpu.sync_copy(ys_hbm.at[batch, pl.ds(0, L)], stage)
    py0 = stage[...][0]
    pltpu.sync_copy(zs_hbm.at[batch, pl.ds(0, L)], stage)
    pz0 = stage[...][0]

    lane = lax.iota(jnp.int32, L)

    @pl.when(part == 0)
    def _():
        row0 = jnp.where(lane == 2, jnp.full((L,), px0, dtype=jnp.float32),
                         jnp.where(lane == 3,
                                   jnp.full((L,), py0, dtype=jnp.float32),
                                   jnp.full((L,), pz0, dtype=jnp.float32)))
        outrows[0] = row0

    gi_base = lane + base
    neginf16 = jnp.full((L,), -jnp.inf, dtype=jnp.float32)
    zero_i16 = jnp.zeros((L,), dtype=jnp.int32)

    def step(i, carry):
        px, py, pz = carry
        pxv = jnp.full((L,), px, dtype=jnp.float32)
        pyv = jnp.full((L,), py, dtype=jnp.float32)
        pzv = jnp.full((L,), pz, dtype=jnp.float32)

        def sweep(off, st):
            bv0, bi0, bv1, bi1 = st
            off1 = off + L
            dx0 = xv[pl.ds(off, L)] - pxv
            dy0 = yv[pl.ds(off, L)] - pyv
            dz0 = zv[pl.ds(off, L)] - pzv
            dx1 = xv[pl.ds(off1, L)] - pxv
            dy1 = yv[pl.ds(off1, L)] - pyv
            dz1 = zv[pl.ds(off1, L)] - pzv
            d0 = dx0 * dx0 + dy0 * dy0 + dz0 * dz0
            d1 = dx1 * dx1 + dy1 * dy1 + dz1 * dz1
            dn0 = jnp.minimum(dv[pl.ds(off, L)], d0)
            dn1 = jnp.minimum(dv[pl.ds(off1, L)], d1)
            dv[pl.ds(off, L)] = dn0
            dv[pl.ds(off1, L)] = dn1
            m0 = dn0 > bv0
            m1 = dn1 > bv1
            bv0 = jnp.where(m0, dn0, bv0)
            bi0 = jnp.where(m0, gi_base + off, bi0)
            bv1 = jnp.where(m1, dn1, bv1)
            bi1 = jnp.where(m1, gi_base + off1, bi1)
            return bv0, bi0, bv1, bi1

        bv0, bi0, bv1, bi1 = plsc.parallel_loop(
            0, NPT, step=2 * L, unroll=4,
            carry=(neginf16, zero_i16, neginf16, zero_i16))(sweep)

        bigv = jnp.full((L,), BIG, dtype=jnp.int32)
        bestv = jnp.maximum(bv0, bv1)
        besti = jnp.minimum(jnp.where(bv0 == bestv, bi0, bigv),
                            jnp.where(bv1 == bestv, bi1, bigv))

        vm = jnp.max(bestv)
        cand = jnp.where(bestv == jnp.full((L,), vm, dtype=jnp.float32),
                         besti, bigv)
        gidx = jnp.min(cand)
        liv = jnp.full((L,), gidx - base, dtype=jnp.int32)
        wx = plsc.load_gather(xv, [liv])
        wy = plsc.load_gather(yv, [liv])
        wz = plsc.load_gather(zv, [liv])

        gf = jnp.full((L,), gidx.astype(jnp.float32), dtype=jnp.float32)
        vmv = jnp.full((L,), vm, dtype=jnp.float32)
        stv = jnp.where(lane == 0, vmv,
                        jnp.where(lane == 1, gf,
                                  jnp.where(lane == 2, wx,
                                            jnp.where(lane == 3, wy, wz))))
        stage[...] = stv

        pltpu.sync_copy(stage, shared.at[s])
        plsc.subcore_barrier()
        pltpu.sync_copy(shared.at[pl.ds(group, TPB)], comb)

        w = comb[0]
        bv = w[0]
        for j in range(1, TPB):
            cj = comb[j]
            v = cj[0]
            better = v > bv
            bv = jnp.where(better, v, bv)
            w = jnp.where(better, cj, w)

        @pl.when(part == 0)
        def _():
            outrows[i] = w

        return w[2], w[3], w[4]

    lax.fori_loop(1, S, step, (px0, py0, pz0))

    @pl.when(part == 0)
    def _():
        pltpu.sync_copy(outrows, out_hbm.at[batch])


_fps_sc = functools.partial(
    pl.kernel,
    out_type=jax.ShapeDtypeStruct((B, S, L), jnp.float32),
    mesh=plsc.VectorSubcoreMesh(core_axis_name="c", subcore_axis_name="s",
                                num_cores=NC, num_subcores=NS),
    scratch_types=[
        pltpu.VMEM((NPT,), jnp.float32),
        pltpu.VMEM((NPT,), jnp.float32),
        pltpu.VMEM((NPT,), jnp.float32),
        pltpu.VMEM((NPT,), jnp.float32),
        pltpu.VMEM((L,), jnp.float32),
        pltpu.VMEM((TPB, L), jnp.float32),
        pltpu.VMEM((S, L), jnp.float32),
        pltpu.VMEM_SHARED((NS, L), jnp.float32),
    ],
    compiler_params=pltpu.CompilerParams(needs_layout_passes=False,
                                         use_tc_tiling_on_sc=False),
)(_fps_body)


@jax.jit
def kernel(point_coord, features):
    del features
    xs = point_coord[:, :, 0]
    ys = point_coord[:, :, 1]
    zs = point_coord[:, :, 2]
    out = _fps_sc(xs, ys, zs)
    return out[:, :, 2:5]


# speedup vs baseline: 28.2906x; 1.0080x over previous
import functools

import jax
import jax.numpy as jnp
from jax import lax
from jax.experimental import pallas as pl
from jax.experimental.pallas import tpu as pltpu
from jax.experimental.pallas import tpu_sc as plsc

B = 8
N = 16384
S = 2048
L = 16
NC = 2
NS = 16
BIG = 1 << 30

TCB = 4
SCB = B - TCB
TPB = (NC * NS) // SCB
NPT = N // TPB
PR = 2
RT = TCB * PR
MT = N // PR


def _fps_sc_body(xs_hbm, ys_hbm, zs_hbm, out_hbm,
                 xv, yv, zv, dv, stage, comb, outrows, shared):
    c = lax.axis_index("c")
    s = lax.axis_index("s")
    batch = c * (SCB // NC) + s // TPB
    part = s % TPB
    base = part * NPT
    group = (s // TPB) * TPB

    pltpu.sync_copy(xs_hbm.at[batch, pl.ds(base, NPT)], xv)
    pltpu.sync_copy(ys_hbm.at[batch, pl.ds(base, NPT)], yv)
    pltpu.sync_copy(zs_hbm.at[batch, pl.ds(base, NPT)], zv)

    inf16 = jnp.full((L,), jnp.inf, dtype=jnp.float32)

    def fill(k, carry):
        dv[pl.ds(k * L, L)] = inf16
        return carry

    lax.fori_loop(0, NPT // L, fill, 0)

    pltpu.sync_copy(xs_hbm.at[batch, pl.ds(0, L)], stage)
    px0 = stage[...][0]
    pltpu.sync_copy(ys_hbm.at[batch, pl.ds(0, L)], stage)
    py0 = stage[...][0]
    pltpu.sync_copy(zs_hbm.at[batch, pl.ds(0, L)], stage)
    pz0 = stage[...][0]

    lane = lax.iota(jnp.int32, L)

    @pl.when(part == 0)
    def _():
        row0 = jnp.where(lane == 2, jnp.full((L,), px0, dtype=jnp.float32),
                         jnp.where(lane == 3,
                                   jnp.full((L,), py0, dtype=jnp.float32),
                                   jnp.full((L,), pz0, dtype=jnp.float32)))
        outrows[0] = row0

    gi_base = lane + base
    neginf16 = jnp.full((L,), -jnp.inf, dtype=jnp.float32)
    zero_i16 = jnp.zeros((L,), dtype=jnp.int32)

    def step(i, carry):
        px, py, pz = carry
        pxv = jnp.full((L,), px, dtype=jnp.float32)
        pyv = jnp.full((L,), py, dtype=jnp.float32)
        pzv = jnp.full((L,), pz, dtype=jnp.float32)

        def sweep(off, st):
            bv0, bi0, bv1, bi1 = st
            off1 = off + L
            dx0 = xv[pl.ds(off, L)] - pxv
            dy0 = yv[pl.ds(off, L)] - pyv
            dz0 = zv[pl.ds(off, L)] - pzv
            dx1 = xv[pl.ds(off1, L)] - pxv
            dy1 = yv[pl.ds(off1, L)] - pyv
            dz1 = zv[pl.ds(off1, L)] - pzv
            d0 = dx0 * dx0 + dz0 * dz0 + dy0 * dy0
            d1 = dx1 * dx1 + dz1 * dz1 + dy1 * dy1
            dn0 = jnp.minimum(dv[pl.ds(off, L)], d0)
            dn1 = jnp.minimum(dv[pl.ds(off1, L)], d1)
            dv[pl.ds(off, L)] = dn0
            dv[pl.ds(off1, L)] = dn1
            m0 = dn0 > bv0
            m1 = dn1 > bv1
            bv0 = jnp.where(m0, dn0, bv0)
            bi0 = jnp.where(m0, gi_base + off, bi0)
            bv1 = jnp.where(m1, dn1, bv1)
            bi1 = jnp.where(m1, gi_base + off1, bi1)
            return bv0, bi0, bv1, bi1

        bv0, bi0, bv1, bi1 = plsc.parallel_loop(
            0, NPT, step=2 * L, unroll=4,
            carry=(neginf16, zero_i16, neginf16, zero_i16))(sweep)

        bigv = jnp.full((L,), BIG, dtype=jnp.int32)
        bestv = jnp.maximum(bv0, bv1)
        besti = jnp.minimum(jnp.where(bv0 == bestv, bi0, bigv),
                            jnp.where(bv1 == bestv, bi1, bigv))

        vm = jnp.max(bestv)
        cand = jnp.where(bestv == jnp.full((L,), vm, dtype=jnp.float32),
                         besti, bigv)
        gidx = jnp.min(cand)
        liv = jnp.full((L,), gidx - base, dtype=jnp.int32)
        wx = plsc.load_gather(xv, [liv])
        wy = plsc.load_gather(yv, [liv])
        wz = plsc.load_gather(zv, [liv])

        gf = jnp.full((L,), gidx.astype(jnp.float32), dtype=jnp.float32)
        vmv = jnp.full((L,), vm, dtype=jnp.float32)
        stv = jnp.where(lane == 0, vmv,
                        jnp.where(lane == 1, gf,
                                  jnp.where(lane == 2, wx,
                                            jnp.where(lane == 3, wy, wz))))
        stage[...] = stv

        pltpu.sync_copy(stage, shared.at[s])
        plsc.subcore_barrier()
        pltpu.sync_copy(shared.at[pl.ds(group, TPB)], comb)

        w = comb[0]
        bv = w[0]
        for j in range(1, TPB):
            cj = comb[j]
            v = cj[0]
            better = v > bv
            bv = jnp.where(better, v, bv)
            w = jnp.where(better, cj, w)

        @pl.when(part == 0)
        def _():
            outrows[i] = w

        return w[2], w[3], w[4]

    lax.fori_loop(1, S, step, (px0, py0, pz0))

    @pl.when(part == 0)
    def _():
        pltpu.sync_copy(outrows, out_hbm.at[batch])


_fps_sc = functools.partial(
    pl.kernel,
    out_type=jax.ShapeDtypeStruct((SCB, S, L), jnp.float32),
    mesh=plsc.VectorSubcoreMesh(core_axis_name="c", subcore_axis_name="s",
                                num_cores=NC, num_subcores=NS),
    scratch_types=[
        pltpu.VMEM((NPT,), jnp.float32),
        pltpu.VMEM((NPT,), jnp.float32),
        pltpu.VMEM((NPT,), jnp.float32),
        pltpu.VMEM((NPT,), jnp.float32),
        pltpu.VMEM((L,), jnp.float32),
        pltpu.VMEM((TPB, L), jnp.float32),
        pltpu.VMEM((S, L), jnp.float32),
        pltpu.VMEM_SHARED((NS, L), jnp.float32),
    ],
    compiler_params=pltpu.CompilerParams(needs_layout_passes=False,
                                         use_tc_tiling_on_sc=False),
)(_fps_sc_body)


def _fps_tc_body(xs_ref, ys_ref, zs_ref, out_ref, dists_ref,
                 ax_ref, ay_ref, az_ref):
    col = lax.broadcasted_iota(jnp.int32, (RT, MT), 1)
    parity = lax.broadcasted_iota(jnp.int32, (RT, 1), 0) % PR
    even = parity == 0
    glob = col + parity * MT
    col128 = lax.broadcasted_iota(jnp.int32, (RT, 128), 1)

    def paircomb(a, op):
        partner = jnp.where(even, pltpu.roll(a, RT - 1, 0),
                            pltpu.roll(a, 1, 0))
        return op(a, partner)

    def coords_at(gr):
        mask = glob == gr
        zero = jnp.zeros((RT, MT), dtype=jnp.float32)
        px = paircomb(jnp.sum(jnp.where(mask, xs_ref[...], zero),
                              axis=1, keepdims=True), jnp.add)
        py = paircomb(jnp.sum(jnp.where(mask, ys_ref[...], zero),
                              axis=1, keepdims=True), jnp.add)
        pz = paircomb(jnp.sum(jnp.where(mask, zs_ref[...], zero),
                              axis=1, keepdims=True), jnp.add)
        return px, py, pz

    def emit(i, px, py, pz):
        ibase = pl.multiple_of((i // 128) * 128, 128)
        m = col128 == (i - ibase)
        ax_ref[:, pl.ds(ibase, 128)] = jnp.where(
            m, px, ax_ref[:, pl.ds(ibase, 128)])
        ay_ref[:, pl.ds(ibase, 128)] = jnp.where(
            m, py, ay_ref[:, pl.ds(ibase, 128)])
        az_ref[:, pl.ds(ibase, 128)] = jnp.where(
            m, pz, az_ref[:, pl.ds(ibase, 128)])

    dists_ref[...] = jnp.full((RT, MT), jnp.inf, dtype=jnp.float32)

    g0 = jnp.zeros((RT, 1), dtype=jnp.int32)
    px, py, pz = coords_at(g0)
    emit(0, px, py, pz)

    def step(i, carry):
        px, py, pz = carry
        dx = xs_ref[...] - px
        dy = ys_ref[...] - py
        dz = zs_ref[...] - pz
        d = dx * dx + dz * dz + dy * dy
        dn = jnp.minimum(dists_ref[...], d)
        dists_ref[...] = dn
        rm = jnp.max(dn, axis=1, keepdims=True)
        bm = paircomb(rm, jnp.maximum)
        cand = jnp.where(dn == bm, glob,
                         jnp.full((RT, MT), BIG, dtype=jnp.int32))
        gi = paircomb(jnp.min(cand, axis=1, keepdims=True), jnp.minimum)
        px, py, pz = coords_at(gi)
        emit(i, px, py, pz)
        return px, py, pz

    lax.fori_loop(1, S, step, (px, py, pz))

    out_ref[0] = ax_ref[...]
    out_ref[1] = ay_ref[...]
    out_ref[2] = az_ref[...]


_fps_tc = pl.pallas_call(
    _fps_tc_body,
    out_shape=jax.ShapeDtypeStruct((3, RT, S), jnp.float32),
    scratch_shapes=[pltpu.VMEM((RT, MT), jnp.float32),
                    pltpu.VMEM((RT, S), jnp.float32),
                    pltpu.VMEM((RT, S), jnp.float32),
                    pltpu.VMEM((RT, S), jnp.float32)],
)


@jax.jit
def kernel(point_coord, features):
    del features
    xs = point_coord[:, :, 0]
    ys = point_coord[:, :, 1]
    zs = point_coord[:, :, 2]
    tc = _fps_tc(xs[:TCB].reshape(RT, MT), ys[:TCB].reshape(RT, MT),
                 zs[:TCB].reshape(RT, MT))
    sc = _fps_sc(xs[TCB:], ys[TCB:], zs[TCB:])
    tc_b = jnp.transpose(tc[:, ::PR, :], (1, 2, 0))
    return jnp.concatenate([tc_b, sc[:, :, 2:5]], axis=0)


# speedup vs baseline: 28.3218x; 1.0011x over previous
import functools

import jax
import jax.numpy as jnp
from jax import lax
from jax.experimental import pallas as pl
from jax.experimental.pallas import tpu as pltpu
from jax.experimental.pallas import tpu_sc as plsc

B = 8
N = 16384
S = 2048
L = 16
NC = 2
NS = 16
BIG = 1 << 30

TCB = 4
SCB = B - TCB
TPB = (NC * NS) // SCB
NPT = N // TPB
PR = 2
RT = TCB * PR
MT = N // PR


def _fps_sc_body(xs_hbm, ys_hbm, zs_hbm, out_hbm,
                 xv, yv, zv, dv, stage, comb, outrows, shared):
    c = lax.axis_index("c")
    s = lax.axis_index("s")
    batch = c * (SCB // NC) + s // TPB
    part = s % TPB
    base = part * NPT
    group = (s // TPB) * TPB

    pltpu.sync_copy(xs_hbm.at[batch, pl.ds(base, NPT)], xv)
    pltpu.sync_copy(ys_hbm.at[batch, pl.ds(base, NPT)], yv)
    pltpu.sync_copy(zs_hbm.at[batch, pl.ds(base, NPT)], zv)

    inf16 = jnp.full((L,), jnp.inf, dtype=jnp.float32)

    def fill(k, carry):
        dv[pl.ds(k * L, L)] = inf16
        return carry

    lax.fori_loop(0, NPT // L, fill, 0)

    pltpu.sync_copy(xs_hbm.at[batch, pl.ds(0, L)], stage)
    px0 = stage[...][0]
    pltpu.sync_copy(ys_hbm.at[batch, pl.ds(0, L)], stage)
    py0 = stage[...][0]
    pltpu.sync_copy(zs_hbm.at[batch, pl.ds(0, L)], stage)
    pz0 = stage[...][0]

    lane = lax.iota(jnp.int32, L)

    @pl.when(part == 0)
    def _():
        row0 = jnp.where(lane == 2, jnp.full((L,), px0, dtype=jnp.float32),
                         jnp.where(lane == 3,
                                   jnp.full((L,), py0, dtype=jnp.float32),
                                   jnp.full((L,), pz0, dtype=jnp.float32)))
        outrows[0] = row0

    gi_base = lane + base
    neginf16 = jnp.full((L,), -jnp.inf, dtype=jnp.float32)
    zero_i16 = jnp.zeros((L,), dtype=jnp.int32)

    def step(i, carry):
        px, py, pz = carry
        pxv = jnp.full((L,), px, dtype=jnp.float32)
        pyv = jnp.full((L,), py, dtype=jnp.float32)
        pzv = jnp.full((L,), pz, dtype=jnp.float32)

        def sweep(off, st):
            bv0, bi0, bv1, bi1 = st
            off1 = off + L
            dx0 = xv[pl.ds(off, L)] - pxv
            dy0 = yv[pl.ds(off, L)] - pyv
            dz0 = zv[pl.ds(off, L)] - pzv
            dx1 = xv[pl.ds(off1, L)] - pxv
            dy1 = yv[pl.ds(off1, L)] - pyv
            dz1 = zv[pl.ds(off1, L)] - pzv
            d0 = dx0 * dx0 + dz0 * dz0 + dy0 * dy0
            d1 = dx1 * dx1 + dz1 * dz1 + dy1 * dy1
            dn0 = jnp.minimum(dv[pl.ds(off, L)], d0)
            dn1 = jnp.minimum(dv[pl.ds(off1, L)], d1)
            dv[pl.ds(off, L)] = dn0
            dv[pl.ds(off1, L)] = dn1
            m0 = dn0 > bv0
            m1 = dn1 > bv1
            bv0 = jnp.where(m0, dn0, bv0)
            bi0 = jnp.where(m0, gi_base + off, bi0)
            bv1 = jnp.where(m1, dn1, bv1)
            bi1 = jnp.where(m1, gi_base + off1, bi1)
            return bv0, bi0, bv1, bi1

        bv0, bi0, bv1, bi1 = plsc.parallel_loop(
            0, NPT, step=2 * L, unroll=4,
            carry=(neginf16, zero_i16, neginf16, zero_i16))(sweep)

        bigv = jnp.full((L,), BIG, dtype=jnp.int32)
        bestv = jnp.maximum(bv0, bv1)
        besti = jnp.minimum(jnp.where(bv0 == bestv, bi0, bigv),
                            jnp.where(bv1 == bestv, bi1, bigv))

        vm = jnp.max(bestv)
        cand = jnp.where(bestv == jnp.full((L,), vm, dtype=jnp.float32),
                         besti, bigv)
        gidx = jnp.min(cand)
        liv = jnp.full((L,), gidx - base, dtype=jnp.int32)
        wx = plsc.load_gather(xv, [liv])
        wy = plsc.load_gather(yv, [liv])
        wz = plsc.load_gather(zv, [liv])

        gf = jnp.full((L,), gidx.astype(jnp.float32), dtype=jnp.float32)
        vmv = jnp.full((L,), vm, dtype=jnp.float32)
        stv = jnp.where(lane == 0, vmv,
                        jnp.where(lane == 1, gf,
                                  jnp.where(lane == 2, wx,
                                            jnp.where(lane == 3, wy, wz))))
        stage[...] = stv

        pltpu.sync_copy(stage, shared.at[s])
        plsc.subcore_barrier()
        pltpu.sync_copy(shared.at[pl.ds(group, TPB)], comb)

        w = comb[0]
        bv = w[0]
        for j in range(1, TPB):
            cj = comb[j]
            v = cj[0]
            better = v > bv
            bv = jnp.where(better, v, bv)
            w = jnp.where(better, cj, w)

        @pl.when(part == 0)
        def _():
            outrows[i] = w

        return w[2], w[3], w[4]

    lax.fori_loop(1, S, step, (px0, py0, pz0))

    @pl.when(part == 0)
    def _():
        pltpu.sync_copy(outrows, out_hbm.at[batch])


_fps_sc = functools.partial(
    pl.kernel,
    out_type=jax.ShapeDtypeStruct((SCB, S, L), jnp.float32),
    mesh=plsc.VectorSubcoreMesh(core_axis_name="c", subcore_axis_name="s",
                                num_cores=NC, num_subcores=NS),
    scratch_types=[
        pltpu.VMEM((NPT,), jnp.float32),
        pltpu.VMEM((NPT,), jnp.float32),
        pltpu.VMEM((NPT,), jnp.float32),
        pltpu.VMEM((NPT,), jnp.float32),
        pltpu.VMEM((L,), jnp.float32),
        pltpu.VMEM((TPB, L), jnp.float32),
        pltpu.VMEM((S, L), jnp.float32),
        pltpu.VMEM_SHARED((NS, L), jnp.float32),
    ],
    compiler_params=pltpu.CompilerParams(needs_layout_passes=False,
                                         use_tc_tiling_on_sc=False),
)(_fps_sc_body)


def _fps_tc_body(xs_ref, ys_ref, zs_ref, out_ref, dists_ref,
                 ax_ref, ay_ref, az_ref):
    col = lax.broadcasted_iota(jnp.int32, (RT, MT), 1)
    parity = lax.broadcasted_iota(jnp.int32, (RT, 1), 0) % PR
    even = parity == 0
    glob = col + parity * MT
    col128 = lax.broadcasted_iota(jnp.int32, (RT, 128), 1)

    def paircomb(a, op):
        partner = jnp.where(even, pltpu.roll(a, RT - 1, 0),
                            pltpu.roll(a, 1, 0))
        return op(a, partner)

    def coords_at(gr):
        mask = glob == gr
        zero = jnp.zeros((RT, MT), dtype=jnp.float32)
        px = paircomb(jnp.sum(jnp.where(mask, xs_ref[...], zero),
                              axis=1, keepdims=True), jnp.add)
        py = paircomb(jnp.sum(jnp.where(mask, ys_ref[...], zero),
                              axis=1, keepdims=True), jnp.add)
        pz = paircomb(jnp.sum(jnp.where(mask, zs_ref[...], zero),
                              axis=1, keepdims=True), jnp.add)
        return px, py, pz

    def emit(i, px, py, pz):
        ibase = pl.multiple_of((i // 128) * 128, 128)
        m = col128 == (i - ibase)
        ax_ref[:, pl.ds(ibase, 128)] = jnp.where(
            m, px, ax_ref[:, pl.ds(ibase, 128)])
        ay_ref[:, pl.ds(ibase, 128)] = jnp.where(
            m, py, ay_ref[:, pl.ds(ibase, 128)])
        az_ref[:, pl.ds(ibase, 128)] = jnp.where(
            m, pz, az_ref[:, pl.ds(ibase, 128)])

    dists_ref[...] = jnp.full((RT, MT), jnp.inf, dtype=jnp.float32)

    g0 = jnp.zeros((RT, 1), dtype=jnp.int32)
    px, py, pz = coords_at(g0)
    emit(0, px, py, pz)

    def step(i, carry):
        px, py, pz = carry
        dx = xs_ref[...] - px
        dy = ys_ref[...] - py
        dz = zs_ref[...] - pz
        d = dx * dx + dz * dz + dy * dy
        dn = jnp.minimum(dists_ref[...], d)
        dists_ref[...] = dn
        rm = jnp.max(dn, axis=1, keepdims=True)
        bm = paircomb(rm, jnp.maximum)
        cand = jnp.where(dn == bm, glob,
                         jnp.full((RT, MT), BIG, dtype=jnp.int32))
        gi = paircomb(jnp.min(cand, axis=1, keepdims=True), jnp.minimum)
        px, py, pz = coords_at(gi)
        emit(i, px, py, pz)
        return px, py, pz

    lax.fori_loop(1, S, step, (px, py, pz))

    out_ref[0] = ax_ref[...]
    out_ref[1] = ay_ref[...]
    out_ref[2] = az_ref[...]


_fps_tc = pl.pallas_call(
    _fps_tc_body,
    out_shape=jax.ShapeDtypeStruct((3, RT, S), jnp.float32),
    scratch_shapes=[pltpu.VMEM((RT, MT), jnp.float32),
                    pltpu.VMEM((RT, S), jnp.float32),
                    pltpu.VMEM((RT, S), jnp.float32),
                    pltpu.VMEM((RT, S), jnp.float32)],
)


@jax.jit
def kernel(point_coord, features):
    del features
    xs = point_coord[:, :, 0]
    ys = point_coord[:, :, 1]
    zs = point_coord[:, :, 2]
    sc = _fps_sc(xs[TCB:], ys[TCB:], zs[TCB:])
    tc = _fps_tc(xs[:TCB].reshape(RT, MT), ys[:TCB].reshape(RT, MT),
                 zs[:TCB].reshape(RT, MT))
    tc_b = jnp.transpose(tc[:, ::PR, :], (1, 2, 0))
    return jnp.concatenate([tc_b, sc[:, :, 2:5]], axis=0)


# speedup vs baseline: 28.3663x; 1.0016x over previous
import functools

import jax
import jax.numpy as jnp
from jax import lax
from jax.experimental import pallas as pl
from jax.experimental.pallas import tpu as pltpu
from jax.experimental.pallas import tpu_sc as plsc

B = 8
N = 16384
S = 2048
L = 16
NC = 2
NS = 16
BIG = 1 << 30

TCB = 4
SCB = B - TCB
TPB = (NC * NS) // SCB
NPT = N // TPB
PR = 2
RT = TCB * PR
MT = N // PR


def _fps_sc_body(xs_hbm, ys_hbm, zs_hbm, out_hbm,
                 xv, yv, zv, dv, stage, comb, outrows, shared):
    c = lax.axis_index("c")
    s = lax.axis_index("s")
    batch = c * (SCB // NC) + s // TPB
    part = s % TPB
    base = part * NPT
    group = (s // TPB) * TPB

    pltpu.sync_copy(xs_hbm.at[batch, pl.ds(base, NPT)], xv)
    pltpu.sync_copy(ys_hbm.at[batch, pl.ds(base, NPT)], yv)
    pltpu.sync_copy(zs_hbm.at[batch, pl.ds(base, NPT)], zv)

    inf16 = jnp.full((L,), jnp.inf, dtype=jnp.float32)

    def fill(k, carry):
        dv[pl.ds(k * L, L)] = inf16
        return carry

    lax.fori_loop(0, NPT // L, fill, 0)

    pltpu.sync_copy(xs_hbm.at[batch, pl.ds(0, L)], stage)
    px0 = stage[...][0]
    pltpu.sync_copy(ys_hbm.at[batch, pl.ds(0, L)], stage)
    py0 = stage[...][0]
    pltpu.sync_copy(zs_hbm.at[batch, pl.ds(0, L)], stage)
    pz0 = stage[...][0]

    lane = lax.iota(jnp.int32, L)

    @pl.when(part == 0)
    def _():
        row0 = jnp.where(lane == 2, jnp.full((L,), px0, dtype=jnp.float32),
                         jnp.where(lane == 3,
                                   jnp.full((L,), py0, dtype=jnp.float32),
                                   jnp.full((L,), pz0, dtype=jnp.float32)))
        outrows[0] = row0

    gi_base = lane + base
    neginf16 = jnp.full((L,), -jnp.inf, dtype=jnp.float32)
    zero_i16 = jnp.zeros((L,), dtype=jnp.int32)

    def step(i, carry):
        px, py, pz = carry
        pxv = jnp.full((L,), px, dtype=jnp.float32)
        pyv = jnp.full((L,), py, dtype=jnp.float32)
        pzv = jnp.full((L,), pz, dtype=jnp.float32)

        def sweep(off, st):
            bv0, bi0, bv1, bi1 = st
            off1 = off + L
            dx0 = xv[pl.ds(off, L)] - pxv
            dy0 = yv[pl.ds(off, L)] - pyv
            dz0 = zv[pl.ds(off, L)] - pzv
            dx1 = xv[pl.ds(off1, L)] - pxv
            dy1 = yv[pl.ds(off1, L)] - pyv
            dz1 = zv[pl.ds(off1, L)] - pzv
            d0 = dx0 * dx0 + dz0 * dz0 + dy0 * dy0
            d1 = dx1 * dx1 + dz1 * dz1 + dy1 * dy1
            dn0 = jnp.minimum(dv[pl.ds(off, L)], d0)
            dn1 = jnp.minimum(dv[pl.ds(off1, L)], d1)
            dv[pl.ds(off, L)] = dn0
            dv[pl.ds(off1, L)] = dn1
            m0 = dn0 > bv0
            m1 = dn1 > bv1
            bv0 = jnp.where(m0, dn0, bv0)
            bi0 = jnp.where(m0, gi_base + off, bi0)
            bv1 = jnp.where(m1, dn1, bv1)
            bi1 = jnp.where(m1, gi_base + off1, bi1)
            return bv0, bi0, bv1, bi1

        bv0, bi0, bv1, bi1 = plsc.parallel_loop(
            0, NPT, step=2 * L, unroll=4,
            carry=(neginf16, zero_i16, neginf16, zero_i16))(sweep)

        bigv = jnp.full((L,), BIG, dtype=jnp.int32)
        bestv = jnp.maximum(bv0, bv1)
        besti = jnp.minimum(jnp.where(bv0 == bestv, bi0, bigv),
                            jnp.where(bv1 == bestv, bi1, bigv))

        vm = jnp.max(bestv)
        cand = jnp.where(bestv == jnp.full((L,), vm, dtype=jnp.float32),
                         besti, bigv)
        gidx = jnp.min(cand)
        liv = jnp.full((L,), gidx - base, dtype=jnp.int32)
        wx = plsc.load_gather(xv, [liv])
        wy = plsc.load_gather(yv, [liv])
        wz = plsc.load_gather(zv, [liv])

        gf = jnp.full((L,), gidx.astype(jnp.float32), dtype=jnp.float32)
        vmv = jnp.full((L,), vm, dtype=jnp.float32)
        stv = jnp.where(lane == 0, vmv,
                        jnp.where(lane == 1, gf,
                                  jnp.where(lane == 2, wx,
                                            jnp.where(lane == 3, wy, wz))))
        stage[...] = stv

        pltpu.sync_copy(stage, shared.at[s])
        plsc.subcore_barrier()
        pltpu.sync_copy(shared.at[pl.ds(group, TPB)], comb)

        w = comb[0]
        bv = w[0]
        for j in range(1, TPB):
            cj = comb[j]
            v = cj[0]
            better = v > bv
            bv = jnp.where(better, v, bv)
            w = jnp.where(better, cj, w)

        @pl.when(part == 0)
        def _():
            outrows[i] = w

        return w[2], w[3], w[4]

    lax.fori_loop(1, S, step, (px0, py0, pz0))

    @pl.when(part == 0)
    def _():
        pltpu.sync_copy(outrows, out_hbm.at[batch])


_fps_sc = functools.partial(
    pl.kernel,
    out_type=jax.ShapeDtypeStruct((SCB, S, L), jnp.float32),
    mesh=plsc.VectorSubcoreMesh(core_axis_name="c", subcore_axis_name="s",
                                num_cores=NC, num_subcores=NS),
    scratch_types=[
        pltpu.VMEM((NPT,), jnp.float32),
        pltpu.VMEM((NPT,), jnp.float32),
        pltpu.VMEM((NPT,), jnp.float32),
        pltpu.VMEM((NPT,), jnp.float32),
        pltpu.VMEM((L,), jnp.float32),
        pltpu.VMEM((TPB, L), jnp.float32),
        pltpu.VMEM((S, L), jnp.float32),
        pltpu.VMEM_SHARED((NS, L), jnp.float32),
    ],
    compiler_params=pltpu.CompilerParams(needs_layout_passes=False,
                                         use_tc_tiling_on_sc=False),
)(_fps_sc_body)


def _fps_tc_body(xs_ref, ys_ref, zs_ref, out_ref, dists_ref,
                 ax_ref, ay_ref, az_ref):
    col = lax.broadcasted_iota(jnp.int32, (RT, MT), 1)
    parity = lax.broadcasted_iota(jnp.int32, (RT, 1), 0) % PR
    even = parity == 0
    glob = col + parity * MT
    col128 = lax.broadcasted_iota(jnp.int32, (RT, 128), 1)

    def paircomb(a, op):
        partner = jnp.where(even, pltpu.roll(a, RT - 1, 0),
                            pltpu.roll(a, 1, 0))
        return op(a, partner)

    def coords_at(gr):
        mask = glob == gr
        zero = jnp.zeros((RT, MT), dtype=jnp.float32)
        px = paircomb(jnp.sum(jnp.where(mask, xs_ref[...], zero),
                              axis=1, keepdims=True), jnp.add)
        py = paircomb(jnp.sum(jnp.where(mask, ys_ref[...], zero),
                              axis=1, keepdims=True), jnp.add)
        pz = paircomb(jnp.sum(jnp.where(mask, zs_ref[...], zero),
                              axis=1, keepdims=True), jnp.add)
        return px, py, pz

    def emit(i, px, py, pz):
        ibase = pl.multiple_of((i // 128) * 128, 128)
        m = col128 == (i - ibase)
        ax_ref[:, pl.ds(ibase, 128)] = jnp.where(
            m, px, ax_ref[:, pl.ds(ibase, 128)])
        ay_ref[:, pl.ds(ibase, 128)] = jnp.where(
            m, py, ay_ref[:, pl.ds(ibase, 128)])
        az_ref[:, pl.ds(ibase, 128)] = jnp.where(
            m, pz, az_ref[:, pl.ds(ibase, 128)])

    dists_ref[...] = jnp.full((RT, MT), jnp.inf, dtype=jnp.float32)

    g0 = jnp.zeros((RT, 1), dtype=jnp.int32)
    px, py, pz = coords_at(g0)
    emit(0, px, py, pz)

    def step(i, carry):
        px, py, pz = carry
        dx = xs_ref[...] - px
        dy = ys_ref[...] - py
        dz = zs_ref[...] - pz
        d = dx * dx + dz * dz + dy * dy
        dn = jnp.minimum(dists_ref[...], d)
        dists_ref[...] = dn
        rm = jnp.max(dn, axis=1, keepdims=True)
        bm = paircomb(rm, jnp.maximum)
        cand = jnp.where(dn == bm, glob,
                         jnp.full((RT, MT), BIG, dtype=jnp.int32))
        gi = paircomb(jnp.min(cand, axis=1, keepdims=True), jnp.minimum)
        px, py, pz = coords_at(gi)
        emit(i, px, py, pz)
        return px, py, pz

    lax.fori_loop(1, S, step, (px, py, pz))

    out_ref[0] = ax_ref[...]
    out_ref[1] = ay_ref[...]
    out_ref[2] = az_ref[...]


_fps_tc = pl.pallas_call(
    _fps_tc_body,
    out_shape=jax.ShapeDtypeStruct((3, RT, S), jnp.float32),
    scratch_shapes=[pltpu.VMEM((RT, MT), jnp.float32),
                    pltpu.VMEM((RT, S), jnp.float32),
                    pltpu.VMEM((RT, S), jnp.float32),
                    pltpu.VMEM((RT, S), jnp.float32)],
    cost_estimate=pl.CostEstimate(flops=3_000_000_000,
                                  bytes_accessed=2_000_000_000,
                                  transcendentals=0),
)


@jax.jit
def kernel(point_coord, features):
    del features
    xs = point_coord[:, :, 0]
    ys = point_coord[:, :, 1]
    zs = point_coord[:, :, 2]
    sc = _fps_sc(xs[TCB:], ys[TCB:], zs[TCB:])
    tc = _fps_tc(xs[:TCB].reshape(RT, MT), ys[:TCB].reshape(RT, MT),
                 zs[:TCB].reshape(RT, MT))
    tc_b = jnp.transpose(tc[:, ::PR, :], (1, 2, 0))
    return jnp.concatenate([tc_b, sc[:, :, 2:5]], axis=0)
